# Initial kernel scaffold; baseline (speedup 1.0000x reference)
import jax
import jax.numpy as jnp
from jax import lax
from jax.experimental import pallas as pl
from jax.experimental.pallas import tpu as pltpu
from jax.experimental.pallas import tpu_sc as plsc

N = 50000
D = 64
HALF = 32
E = 800000
NS = 16
CS = 128
NB = 8
G = 49
EDGES_PER_TILE = G * NB * CS
E_PAD = NS * EDGES_PER_TILE
CH_TOT = E_PAD // CS
PAD_ROWS = 64
ACC_ROWS = N + PAD_ROWS
ROWS_PER_TILE = N // NS

BN = 1000
NBLK = N // BN


def _tc_body(x_ref, w_ref, b_ref, out_ref):
    y = jnp.dot(x_ref[...], w_ref[0], preferred_element_type=jnp.float32)
    y = y + b_ref[0]
    out_ref[0, 0] = y[:, :HALF]
    out_ref[0, 1] = y[:, HALF:]


def _tc_transform(x, wstack, bstack):
    k = wstack.shape[0]
    return pl.pallas_call(
        _tc_body,
        grid=(k, NBLK),
        in_specs=[
            pl.BlockSpec((BN, D), lambda j, n: (n, 0)),
            pl.BlockSpec((1, D, D), lambda j, n: (j, 0, 0)),
            pl.BlockSpec((1, 1, D), lambda j, n: (j, 0, 0)),
        ],
        out_specs=pl.BlockSpec((1, 2, BN, HALF), lambda j, n: (j, 0, n, 0)),
        out_shape=jax.ShapeDtypeStruct((k, 2, N, HALF), jnp.float32),
    )(x, wstack, bstack)


def _sc_body(init_uav, init_car, init_poi,
             m_uav_car, m_uav_poi, m_car_uav, m_poi_uav,
             d_uc, s_uc, d_up, s_up, d_cu, s_cu, d_pu, s_pu,
             out_uav, out_car, out_poi,
             acc, idx_s, idx_d, rows, gsem, ssem):
    c = lax.axis_index("c")
    s = lax.axis_index("s")
    row0 = s * ROWS_PER_TILE
    chunk0 = s * (G * NB)

    def run_relation(dst2d, src2d, y_hbm):
        yc = y_hbm.at[c]

        def load_and_gather(g):
            slot = lax.rem(g, 2)
            r0 = chunk0 + g * NB
            pltpu.sync_copy(src2d.at[pl.ds(r0, NB)], idx_s.at[slot])
            pltpu.sync_copy(dst2d.at[pl.ds(r0, NB)], idx_d.at[slot])
            for b in range(NB):
                pltpu.async_copy(yc.at[idx_s.at[slot, b]], rows.at[slot, b],
                                 gsem)

        def drain_gather_fire_scatter(g):
            slot = lax.rem(g, 2)
            for b in range(NB):
                pltpu.make_async_copy(yc.at[idx_s.at[slot, b]],
                                      rows.at[slot, b], gsem).wait()
            for b in range(NB):
                pltpu.async_copy(rows.at[slot, b], acc.at[idx_d.at[slot, b]],
                                 ssem, add=True)

        def drain_scatter(g):
            slot = lax.rem(g, 2)
            for b in range(NB):
                pltpu.make_async_copy(rows.at[slot, b],
                                      acc.at[idx_d.at[slot, b]], ssem).wait()

        def body(g, carry):
            @pl.when(g >= 2)
            def _():
                drain_scatter(g - 2)

            @pl.when(g < G)
            def _():
                load_and_gather(g)

            @pl.when(g >= 1)
            def _():
                drain_gather_fire_scatter(g - 1)

            return carry

        lax.fori_loop(0, G + 1, body, 0)
        drain_scatter(G - 1)

    def run_phase(init_hbm, rels, out_hbm):
        pltpu.sync_copy(init_hbm.at[c, pl.ds(row0, ROWS_PER_TILE)],
                        acc.at[pl.ds(row0, ROWS_PER_TILE)])
        plsc.subcore_barrier()
        for dst2d, src2d, y_hbm in rels:
            run_relation(dst2d, src2d, y_hbm)
        plsc.subcore_barrier()
        pltpu.sync_copy(acc.at[pl.ds(row0, ROWS_PER_TILE)],
                        out_hbm.at[c, pl.ds(row0, ROWS_PER_TILE)])

    run_phase(init_uav, [(d_uc, s_uc, m_uav_car), (d_up, s_up, m_uav_poi)],
              out_uav)
    run_phase(init_car, [(d_cu, s_cu, m_car_uav)], out_car)
    run_phase(init_poi, [(d_pu, s_pu, m_poi_uav)], out_poi)


_sds = jax.ShapeDtypeStruct

_sc_call = pl.kernel(
    _sc_body,
    out_type=(_sds((2, N, HALF), jnp.float32),) * 3,
    mesh=plsc.VectorSubcoreMesh(core_axis_name="c", subcore_axis_name="s"),
    scratch_types=[
        pltpu.VMEM_SHARED((ACC_ROWS, HALF), jnp.float32),
        pltpu.VMEM((2, NB, CS), jnp.int32),
        pltpu.VMEM((2, NB, CS), jnp.int32),
        pltpu.VMEM((2, NB, CS, HALF), jnp.float32),
        pltpu.SemaphoreType.DMA,
        pltpu.SemaphoreType.DMA,
    ],
)


def kernel(x_uav, x_carrier, x_poi,
           edge_uav_carrier, edge_uav_poi, edge_carrier_uav, edge_poi_uav,
           w_self_uav, W_uav_carrier, W_uav_poi,
           w_self_carrier, W_carrier_uav,
           w_self_poi, W_poi_uav,
           b_uav, b_carrier, b_poi):
    zb = jnp.zeros((1, D), jnp.float32)
    w_u = jnp.stack([w_self_uav / 3.0, W_carrier_uav / 2.0, W_poi_uav / 2.0])
    b_u = jnp.stack([b_uav, zb, zb])
    w_c = jnp.stack([w_self_carrier / 2.0, W_uav_carrier / 3.0])
    b_c = jnp.stack([b_carrier, zb])
    w_p = jnp.stack([w_self_poi / 2.0, W_uav_poi / 3.0])
    b_p = jnp.stack([b_poi, zb])

    t_u = _tc_transform(x_uav, w_u, b_u)
    t_c = _tc_transform(x_carrier, w_c, b_c)
    t_p = _tc_transform(x_poi, w_p, b_p)

    pad = E_PAD - E
    pad_dst = N + (jnp.arange(pad, dtype=jnp.int32) % PAD_ROWS)
    pad_src = (jnp.arange(pad, dtype=jnp.int32) * 997) % N

    def prep(e):
        dst = jnp.concatenate([e[0], pad_dst]).reshape(CH_TOT, CS)
        src = jnp.concatenate([e[1], pad_src]).reshape(CH_TOT, CS)
        return dst, src

    d_uc, s_uc = prep(edge_uav_carrier)
    d_up, s_up = prep(edge_uav_poi)
    d_cu, s_cu = prep(edge_carrier_uav)
    d_pu, s_pu = prep(edge_poi_uav)

    out_uav, out_car, out_poi = _sc_call(
        t_u[0], t_c[0], t_p[0],
        t_c[1], t_p[1], t_u[1], t_u[2],
        d_uc, s_uc, d_up, s_up, d_cu, s_cu, d_pu, s_pu)

    fin = lambda o: jnp.concatenate([o[0], o[1]], axis=1)
    return (fin(out_uav), fin(out_car), fin(out_poi))


# speedup vs baseline: 5.0544x; 5.0544x over previous
import jax
import jax.numpy as jnp
from jax import lax
from jax.experimental import pallas as pl
from jax.experimental.pallas import tpu as pltpu
from jax.experimental.pallas import tpu_sc as plsc

N = 50000
D = 64
HALF = 32
E = 800000
NS = 16
CS = 128
NB = 3
G = 131
EDGES_PER_TILE = G * NB * CS
E_PAD = NS * EDGES_PER_TILE
CH_TOT = E_PAD // CS
PAD_ROWS = 64
ACC_ROWS = N + PAD_ROWS
RPT = 3128
RPT_LAST = N - 15 * RPT

BN = 1000
NBLK = N // BN


def _tc_body(x_ref, w_ref, b_ref, out_ref):
    y = jnp.dot(x_ref[...], w_ref[0], preferred_element_type=jnp.float32)
    y = y + b_ref[0]
    out_ref[0, 0] = y[:, :HALF]
    out_ref[0, 1] = y[:, HALF:]


def _tc_transform(x, wstack, bstack):
    k = wstack.shape[0]
    return pl.pallas_call(
        _tc_body,
        grid=(k, NBLK),
        in_specs=[
            pl.BlockSpec((BN, D), lambda j, n: (n, 0)),
            pl.BlockSpec((1, D, D), lambda j, n: (j, 0, 0)),
            pl.BlockSpec((1, 1, D), lambda j, n: (j, 0, 0)),
        ],
        out_specs=pl.BlockSpec((1, 2, BN, HALF), lambda j, n: (j, 0, n, 0)),
        out_shape=jax.ShapeDtypeStruct((k, 2, N, HALF), jnp.float32),
    )(x, wstack, bstack)


def _sc_body(init_uav, init_car, init_poi,
             m_uav_car, m_uav_poi, m_car_uav, m_poi_uav,
             d_uc, s_uc, d_up, s_up, d_cu, s_cu, d_pu, s_pu,
             out_uav, out_car, out_poi,
             acc, idx_s, idx_d, rows, gsem, ssem):
    c = lax.axis_index("c")
    s = lax.axis_index("s")
    row0 = s * RPT
    chunk0 = s * (G * NB)

    def stripe_copy(src_of, dst_of):
        @pl.when(s < NS - 1)
        def _():
            pltpu.sync_copy(src_of(row0, RPT), dst_of(row0, RPT))

        @pl.when(s == NS - 1)
        def _():
            pltpu.sync_copy(src_of(row0, RPT_LAST), dst_of(row0, RPT_LAST))

    def run_relation(dst2d, src2d, y_hbm):
        yc = y_hbm.at[c]

        def load_and_gather(g):
            slot = lax.rem(g, 2)
            r0 = chunk0 + g * NB
            pltpu.sync_copy(src2d.at[pl.ds(r0, NB)], idx_s.at[slot])
            pltpu.sync_copy(dst2d.at[pl.ds(r0, NB)], idx_d.at[slot])
            for b in range(NB):
                pltpu.async_copy(yc.at[idx_s.at[slot, b]], rows.at[slot, b],
                                 gsem)

        def drain_gather_fire_scatter(g):
            slot = lax.rem(g, 2)
            for b in range(NB):
                pltpu.make_async_copy(yc.at[idx_s.at[slot, b]],
                                      rows.at[slot, b], gsem).wait()
            for b in range(NB):
                pltpu.async_copy(rows.at[slot, b], acc.at[idx_d.at[slot, b]],
                                 ssem, add=True)

        def drain_scatter(g):
            slot = lax.rem(g, 2)
            for b in range(NB):
                pltpu.make_async_copy(rows.at[slot, b],
                                      acc.at[idx_d.at[slot, b]], ssem).wait()

        def body(g, carry):
            @pl.when(g >= 2)
            def _():
                drain_scatter(g - 2)

            @pl.when(g < G)
            def _():
                load_and_gather(g)

            @pl.when(g >= 1)
            def _():
                drain_gather_fire_scatter(g - 1)

            return carry

        lax.fori_loop(0, G + 1, body, 0)
        drain_scatter(G - 1)

    def run_phase(init_hbm, rels, out_hbm):
        stripe_copy(lambda r, n: init_hbm.at[c, pl.ds(r, n)],
                    lambda r, n: acc.at[pl.ds(r, n)])
        plsc.subcore_barrier()
        for dst2d, src2d, y_hbm in rels:
            run_relation(dst2d, src2d, y_hbm)
        plsc.subcore_barrier()
        stripe_copy(lambda r, n: acc.at[pl.ds(r, n)],
                    lambda r, n: out_hbm.at[c, pl.ds(r, n)])

    run_phase(init_uav, [(d_uc, s_uc, m_uav_car), (d_up, s_up, m_uav_poi)],
              out_uav)
    run_phase(init_car, [(d_cu, s_cu, m_car_uav)], out_car)
    run_phase(init_poi, [(d_pu, s_pu, m_poi_uav)], out_poi)


_sds = jax.ShapeDtypeStruct

_sc_call = pl.kernel(
    _sc_body,
    out_type=(_sds((2, N, HALF), jnp.float32),) * 3,
    mesh=plsc.VectorSubcoreMesh(core_axis_name="c", subcore_axis_name="s"),
    compiler_params=pltpu.CompilerParams(use_tc_tiling_on_sc=False),
    scratch_types=[
        pltpu.VMEM_SHARED((ACC_ROWS, HALF), jnp.float32),
        pltpu.VMEM((2, NB, CS), jnp.int32),
        pltpu.VMEM((2, NB, CS), jnp.int32),
        pltpu.VMEM((2, NB, CS, HALF), jnp.float32),
        pltpu.SemaphoreType.DMA,
        pltpu.SemaphoreType.DMA,
    ],
)


def kernel(x_uav, x_carrier, x_poi,
           edge_uav_carrier, edge_uav_poi, edge_carrier_uav, edge_poi_uav,
           w_self_uav, W_uav_carrier, W_uav_poi,
           w_self_carrier, W_carrier_uav,
           w_self_poi, W_poi_uav,
           b_uav, b_carrier, b_poi):
    zb = jnp.zeros((1, D), jnp.float32)
    w_u = jnp.stack([w_self_uav / 3.0, W_carrier_uav / 2.0, W_poi_uav / 2.0])
    b_u = jnp.stack([b_uav, zb, zb])
    w_c = jnp.stack([w_self_carrier / 2.0, W_uav_carrier / 3.0])
    b_c = jnp.stack([b_carrier, zb])
    w_p = jnp.stack([w_self_poi / 2.0, W_uav_poi / 3.0])
    b_p = jnp.stack([b_poi, zb])

    t_u = _tc_transform(x_uav, w_u, b_u)
    t_c = _tc_transform(x_carrier, w_c, b_c)
    t_p = _tc_transform(x_poi, w_p, b_p)

    pad = E_PAD - E
    pad_dst = N + (jnp.arange(pad, dtype=jnp.int32) % PAD_ROWS)
    pad_src = (jnp.arange(pad, dtype=jnp.int32) * 997) % N

    def prep(e):
        dst = jnp.concatenate([e[0], pad_dst]).reshape(CH_TOT, CS)
        src = jnp.concatenate([e[1], pad_src]).reshape(CH_TOT, CS)
        return dst, src

    d_uc, s_uc = prep(edge_uav_carrier)
    d_up, s_up = prep(edge_uav_poi)
    d_cu, s_cu = prep(edge_carrier_uav)
    d_pu, s_pu = prep(edge_poi_uav)

    out_uav, out_car, out_poi = _sc_call(
        t_u[0], t_c[0], t_p[0],
        t_c[1], t_p[1], t_u[1], t_u[2],
        d_uc, s_uc, d_up, s_up, d_cu, s_cu, d_pu, s_pu)

    fin = lambda o: jnp.concatenate([o[0], o[1]], axis=1)
    return (fin(out_uav), fin(out_car), fin(out_poi))


# speedup vs baseline: 9.6159x; 1.9025x over previous
import jax
import jax.numpy as jnp
from jax import lax
from jax.experimental import pallas as pl
from jax.experimental.pallas import tpu as pltpu
from jax.experimental.pallas import tpu_sc as plsc

N = 50000
D = 64
HALF = 32
E = 800000
NS = 16
CS = 128
NB = 3
G = 131
EDGES_PER_TILE = G * NB * CS
E_PAD = NS * EDGES_PER_TILE
CH_TOT = E_PAD // CS
PAD_ROWS = 64
ACC_ROWS = N + PAD_ROWS
RPT = 3128
RPT_LAST = N - 15 * RPT

NROW4 = N // 4
NP4 = 12800
NPAD = NP4 * 4
BNP = 800
NBLK4 = NP4 // BNP


def _tc_body(x4_ref, w4_ref, b4_ref, out_ref):
    y = jnp.dot(x4_ref[...], w4_ref[...], preferred_element_type=jnp.float32)
    y = y + b4_ref[...]
    for k in range(out_ref.shape[0]):
        out_ref[k] = y[:, k * 128:(k + 1) * 128]


def _tc_transform(x, mats, bias):
    T = len(mats)
    K = 2 * T
    Wr = jnp.stack(mats).reshape(T, D, 2, HALF)
    eye4 = jnp.eye(4, dtype=jnp.float32)
    w4 = jnp.einsum('jJ,tkhc->jkthJc', eye4, Wr).reshape(4 * D, T * 4 * D)
    bz = jnp.zeros((T, D), jnp.float32).at[0].set(bias)
    b4 = jnp.broadcast_to(bz.reshape(T, 2, 1, HALF),
                          (T, 2, 4, HALF)).reshape(1, T * 4 * D)
    x4 = jnp.pad(x.reshape(NROW4, 4 * D), ((0, NP4 - NROW4), (0, 0)))
    out = pl.pallas_call(
        _tc_body,
        grid=(NBLK4,),
        in_specs=[
            pl.BlockSpec((BNP, 4 * D), lambda n: (n, 0)),
            pl.BlockSpec((4 * D, K * 128), lambda n: (0, 0)),
            pl.BlockSpec((1, K * 128), lambda n: (0, 0)),
        ],
        out_specs=pl.BlockSpec((K, BNP, 128), lambda n: (0, n, 0)),
        out_shape=jax.ShapeDtypeStruct((K, NP4, 128), jnp.float32),
    )(x4, w4, b4)
    return out.reshape(K, NPAD, HALF)


def _sc_body(yu, ycar, ypoi,
             e_uc, e_up, e_cu, e_pu,
             out_uav, out_car, out_poi,
             acc, idx_s, idx_d, rows, isem, gsem, ssem):
    c = lax.axis_index("c")
    s = lax.axis_index("s")
    row0 = s * RPT
    chunk0 = s * (G * NB)

    def stripe_copy(src_of, dst_of):
        @pl.when(s < NS - 1)
        def _():
            pltpu.sync_copy(src_of(row0, RPT), dst_of(row0, RPT))

        @pl.when(s == NS - 1)
        def _():
            pltpu.sync_copy(src_of(row0, RPT_LAST), dst_of(row0, RPT_LAST))

    def run_relation(e3d, ytab):
        dst2d = e3d.at[0]
        src2d = e3d.at[1]

        def fire_idx(g):
            slot = lax.rem(g, 3)
            r0 = chunk0 + g * NB
            pltpu.async_copy(src2d.at[pl.ds(r0, NB)], idx_s.at[slot], isem)
            pltpu.async_copy(dst2d.at[pl.ds(r0, NB)], idx_d.at[slot], isem)

        def wait_idx(g):
            slot = lax.rem(g, 3)
            r0 = chunk0 + g * NB
            pltpu.make_async_copy(src2d.at[pl.ds(r0, NB)], idx_s.at[slot],
                                  isem).wait()
            pltpu.make_async_copy(dst2d.at[pl.ds(r0, NB)], idx_d.at[slot],
                                  isem).wait()

        def fire_gather(g):
            islot = lax.rem(g, 3)
            rslot = lax.rem(g, 2)
            for b in range(NB):
                pltpu.async_copy(ytab.at[idx_s.at[islot, b]],
                                 rows.at[rslot, b], gsem)

        def drain_gather_fire_scatter(g):
            islot = lax.rem(g, 3)
            rslot = lax.rem(g, 2)
            for b in range(NB):
                pltpu.make_async_copy(ytab.at[idx_s.at[islot, b]],
                                      rows.at[rslot, b], gsem).wait()
            for b in range(NB):
                pltpu.async_copy(rows.at[rslot, b],
                                 acc.at[idx_d.at[islot, b]], ssem, add=True)

        def drain_scatter(g):
            islot = lax.rem(g, 3)
            rslot = lax.rem(g, 2)
            for b in range(NB):
                pltpu.make_async_copy(rows.at[rslot, b],
                                      acc.at[idx_d.at[islot, b]], ssem).wait()

        fire_idx(0)

        def body(g, carry):
            @pl.when(g >= 2)
            def _():
                drain_scatter(g - 2)

            @pl.when(g < G)
            def _():
                wait_idx(g)

                @pl.when(g + 1 < G)
                def _():
                    fire_idx(g + 1)

                fire_gather(g)

            @pl.when(g >= 1)
            def _():
                drain_gather_fire_scatter(g - 1)

            return carry

        lax.fori_loop(0, G + 1, body, 0)
        drain_scatter(G - 1)

    def run_phase(init_tab, rels, out_hbm):
        stripe_copy(lambda r, n: init_tab.at[pl.ds(r, n)],
                    lambda r, n: acc.at[pl.ds(r, n)])
        plsc.subcore_barrier()
        for e3d, ytab in rels:
            run_relation(e3d, ytab)
        plsc.subcore_barrier()
        stripe_copy(lambda r, n: acc.at[pl.ds(r, n)],
                    lambda r, n: out_hbm.at[pl.ds(r, n), c])

    run_phase(yu.at[c],
              [(e_uc, ycar.at[2 + c]), (e_up, ypoi.at[2 + c])],
              out_uav)
    run_phase(ycar.at[c], [(e_cu, yu.at[2 + c])], out_car)
    run_phase(ypoi.at[c], [(e_pu, yu.at[4 + c])], out_poi)


_sds = jax.ShapeDtypeStruct

_sc_call = pl.kernel(
    _sc_body,
    out_type=(_sds((N, 2, HALF), jnp.float32),) * 3,
    mesh=plsc.VectorSubcoreMesh(core_axis_name="c", subcore_axis_name="s"),
    compiler_params=pltpu.CompilerParams(use_tc_tiling_on_sc=False),
    scratch_types=[
        pltpu.VMEM_SHARED((ACC_ROWS, HALF), jnp.float32),
        pltpu.VMEM((3, NB, CS), jnp.int32),
        pltpu.VMEM((3, NB, CS), jnp.int32),
        pltpu.VMEM((2, NB, CS, HALF), jnp.float32),
        pltpu.SemaphoreType.DMA,
        pltpu.SemaphoreType.DMA,
        pltpu.SemaphoreType.DMA,
    ],
)


def kernel(x_uav, x_carrier, x_poi,
           edge_uav_carrier, edge_uav_poi, edge_carrier_uav, edge_poi_uav,
           w_self_uav, W_uav_carrier, W_uav_poi,
           w_self_carrier, W_carrier_uav,
           w_self_poi, W_poi_uav,
           b_uav, b_carrier, b_poi):
    yu = _tc_transform(x_uav,
                       [w_self_uav / 3.0, W_carrier_uav / 2.0,
                        W_poi_uav / 2.0], b_uav.reshape(D))
    ycar = _tc_transform(x_carrier,
                         [w_self_carrier / 2.0, W_uav_carrier / 3.0],
                         b_carrier.reshape(D))
    ypoi = _tc_transform(x_poi,
                         [w_self_poi / 2.0, W_uav_poi / 3.0],
                         b_poi.reshape(D))

    pad = E_PAD - E
    pad_pair = jnp.stack([
        N + (jnp.arange(pad, dtype=jnp.int32) % PAD_ROWS),
        (jnp.arange(pad, dtype=jnp.int32) * 997) % N,
    ])

    def prep(e):
        return jnp.concatenate([e, pad_pair], axis=1).reshape(2, CH_TOT, CS)

    e_uc = prep(edge_uav_carrier)
    e_up = prep(edge_uav_poi)
    e_cu = prep(edge_carrier_uav)
    e_pu = prep(edge_poi_uav)

    out_uav, out_car, out_poi = _sc_call(
        yu, ycar, ypoi, e_uc, e_up, e_cu, e_pu)

    return (out_uav.reshape(N, D), out_car.reshape(N, D),
            out_poi.reshape(N, D))


# speedup vs baseline: 9.6379x; 1.0023x over previous
import jax
import jax.numpy as jnp
from jax import lax
from jax.experimental import pallas as pl
from jax.experimental.pallas import tpu as pltpu
from jax.experimental.pallas import tpu_sc as plsc

N = 50000
D = 64
HALF = 32
E = 800000
NS = 16
CS = 128
NB = 3
G = 131
EDGES_PER_TILE = G * NB * CS
E_PAD = NS * EDGES_PER_TILE
CH_TOT = E_PAD // CS
PAD_ROWS = 64
ACC_ROWS = N + PAD_ROWS
RPT = 3128
RPT_LAST = N - 15 * RPT

NROW4 = N // 4
NP4 = 12800
NPAD = NP4 * 4
BNP = 800
NBLK4 = NP4 // BNP


def _tc_body(x4_ref, w4_ref, b4_ref, out_ref):
    y = jnp.dot(x4_ref[...], w4_ref[...], preferred_element_type=jnp.float32)
    y = y + b4_ref[...]
    for k in range(out_ref.shape[0]):
        out_ref[k] = y[:, k * 128:(k + 1) * 128]


def _tc_transform(x, mats, bias):
    T = len(mats)
    K = 2 * T
    Wr = jnp.stack(mats).reshape(T, D, 2, HALF)
    eye4 = jnp.eye(4, dtype=jnp.float32)
    w4 = jnp.einsum('jJ,tkhc->jkthJc', eye4, Wr).reshape(4 * D, T * 4 * D)
    bz = jnp.zeros((T, D), jnp.float32).at[0].set(bias)
    b4 = jnp.broadcast_to(bz.reshape(T, 2, 1, HALF),
                          (T, 2, 4, HALF)).reshape(1, T * 4 * D)
    x4 = jnp.pad(x.reshape(NROW4, 4 * D), ((0, NP4 - NROW4), (0, 0)))
    out = pl.pallas_call(
        _tc_body,
        grid=(NBLK4,),
        in_specs=[
            pl.BlockSpec((BNP, 4 * D), lambda n: (n, 0)),
            pl.BlockSpec((4 * D, K * 128), lambda n: (0, 0)),
            pl.BlockSpec((1, K * 128), lambda n: (0, 0)),
        ],
        out_specs=pl.BlockSpec((K, BNP, 128), lambda n: (0, n, 0)),
        out_shape=jax.ShapeDtypeStruct((K, NP4, 128), jnp.float32),
    )(x4, w4, b4)
    return out.reshape(K, NPAD, HALF)


def _sc_phase(acc, idx_s, idx_d, rows, isem, gsem, ssem,
              init_tab, rels, out_hbm, c, s):
    row0 = s * RPT
    chunk0 = s * (G * NB)

    def stripe_copy(src_of, dst_of):
        @pl.when(s < NS - 1)
        def _():
            pltpu.sync_copy(src_of(row0, RPT), dst_of(row0, RPT))

        @pl.when(s == NS - 1)
        def _():
            pltpu.sync_copy(src_of(row0, RPT_LAST), dst_of(row0, RPT_LAST))

    def run_relation(e3d, ytab):
        dst2d = e3d.at[0]
        src2d = e3d.at[1]

        def fire_idx(g):
            slot = lax.rem(g, 3)
            r0 = chunk0 + g * NB
            pltpu.async_copy(src2d.at[pl.ds(r0, NB)], idx_s.at[slot], isem)
            pltpu.async_copy(dst2d.at[pl.ds(r0, NB)], idx_d.at[slot], isem)

        def wait_idx(g):
            slot = lax.rem(g, 3)
            r0 = chunk0 + g * NB
            pltpu.make_async_copy(src2d.at[pl.ds(r0, NB)], idx_s.at[slot],
                                  isem).wait()
            pltpu.make_async_copy(dst2d.at[pl.ds(r0, NB)], idx_d.at[slot],
                                  isem).wait()

        def fire_gather(g):
            islot = lax.rem(g, 3)
            rslot = lax.rem(g, 2)
            for b in range(NB):
                pltpu.async_copy(ytab.at[idx_s.at[islot, b]],
                                 rows.at[rslot, b], gsem)

        def drain_gather_fire_scatter(g):
            islot = lax.rem(g, 3)
            rslot = lax.rem(g, 2)
            for b in range(NB):
                pltpu.make_async_copy(ytab.at[idx_s.at[islot, b]],
                                      rows.at[rslot, b], gsem).wait()
            for b in range(NB):
                pltpu.async_copy(rows.at[rslot, b],
                                 acc.at[idx_d.at[islot, b]], ssem, add=True)

        def drain_scatter(g):
            islot = lax.rem(g, 3)
            rslot = lax.rem(g, 2)
            for b in range(NB):
                pltpu.make_async_copy(rows.at[rslot, b],
                                      acc.at[idx_d.at[islot, b]], ssem).wait()

        fire_idx(0)

        def body(g, carry):
            @pl.when(g >= 2)
            def _():
                drain_scatter(g - 2)

            @pl.when(g < G)
            def _():
                wait_idx(g)

                @pl.when(g + 1 < G)
                def _():
                    fire_idx(g + 1)

                fire_gather(g)

            @pl.when(g >= 1)
            def _():
                drain_gather_fire_scatter(g - 1)

            return carry

        lax.fori_loop(0, G + 1, body, 0)
        drain_scatter(G - 1)

    stripe_copy(lambda r, n: init_tab.at[pl.ds(r, n)],
                lambda r, n: acc.at[pl.ds(r, n)])
    plsc.subcore_barrier()
    for e3d, ytab in rels:
        run_relation(e3d, ytab)
    plsc.subcore_barrier()
    stripe_copy(lambda r, n: acc.at[pl.ds(r, n)],
                lambda r, n: out_hbm.at[pl.ds(r, n), c])


def _sc_fused_body(yu, ycar, ypoi, e_uc, e_up, e_cu, e_pu,
                   out_u, out_c, out_p, *scratch):
    c = lax.axis_index("c")
    s = lax.axis_index("s")
    _sc_phase(*scratch, yu.at[c],
              [(e_uc, ycar.at[2 + c]), (e_up, ypoi.at[2 + c])], out_u, c, s)
    _sc_phase(*scratch, ycar.at[c], [(e_cu, yu.at[2 + c])], out_c, c, s)
    _sc_phase(*scratch, ypoi.at[c], [(e_pu, yu.at[4 + c])], out_p, c, s)


_sds = jax.ShapeDtypeStruct
_sc_call = pl.kernel(
    _sc_fused_body,
    out_type=(_sds((N, 2, HALF), jnp.float32),
              _sds((N, 2, HALF), jnp.float32),
              _sds((N, 2, HALF), jnp.float32)),
    mesh=plsc.VectorSubcoreMesh(core_axis_name="c", subcore_axis_name="s"),
    compiler_params=pltpu.CompilerParams(use_tc_tiling_on_sc=False),
    scratch_types=[
        pltpu.VMEM_SHARED((ACC_ROWS, HALF), jnp.float32),
        pltpu.VMEM((3, NB, CS), jnp.int32),
        pltpu.VMEM((3, NB, CS), jnp.int32),
        pltpu.VMEM((2, NB, CS, HALF), jnp.float32),
        pltpu.SemaphoreType.DMA,
        pltpu.SemaphoreType.DMA,
        pltpu.SemaphoreType.DMA,
    ],
)


def kernel(x_uav, x_carrier, x_poi,
           edge_uav_carrier, edge_uav_poi, edge_carrier_uav, edge_poi_uav,
           w_self_uav, W_uav_carrier, W_uav_poi,
           w_self_carrier, W_carrier_uav,
           w_self_poi, W_poi_uav,
           b_uav, b_carrier, b_poi):
    yu = _tc_transform(x_uav,
                       [w_self_uav / 3.0, W_carrier_uav / 2.0,
                        W_poi_uav / 2.0], b_uav.reshape(D))
    ycar = _tc_transform(x_carrier,
                         [w_self_carrier / 2.0, W_uav_carrier / 3.0],
                         b_carrier.reshape(D))
    ypoi = _tc_transform(x_poi,
                         [w_self_poi / 2.0, W_uav_poi / 3.0],
                         b_poi.reshape(D))

    pad = E_PAD - E
    pad_pair = jnp.stack([
        N + (jnp.arange(pad, dtype=jnp.int32) % PAD_ROWS),
        (jnp.arange(pad, dtype=jnp.int32) * 997) % N,
    ])

    def prep(e):
        return jnp.concatenate([e, pad_pair], axis=1).reshape(2, CH_TOT, CS)

    e_uc = prep(edge_uav_carrier)
    e_up = prep(edge_uav_poi)
    e_cu = prep(edge_carrier_uav)
    e_pu = prep(edge_poi_uav)

    out_uav, out_car, out_poi = _sc_call(
        yu, ycar, ypoi, e_uc, e_up, e_cu, e_pu)

    return (out_uav.reshape(N, D), out_car.reshape(N, D),
            out_poi.reshape(N, D))
